# Initial kernel scaffold; baseline (speedup 1.0000x reference)
import jax
import jax.numpy as jnp
from jax.experimental import pallas as pl


def kernel(x, edge_index, edge_attr, l1_w1, l1_b1, l1_w2, l1_b2, l1_root, l1_bias, l2_w1, l2_b1, l2_w2, l2_b2, l2_root, l2_bias):
    raise NotImplementedError("write your pallas kernel here")


# speedup vs baseline: 2.1613x; 2.1613x over previous
import functools

import jax
import jax.numpy as jnp
from jax import lax
from jax.experimental import pallas as pl
from jax.experimental.pallas import tpu as pltpu
from jax.experimental.pallas import tpu_sc as plsc

N = 10000
E = 160000
IN = 128
HID = 8

NC = 2
NS = 16
NW = NC * NS
CHUNK = 128
E_PAD = 163840
EPW = E_PAD // NW
NCHUNK = EPW // CHUNK
N_PAD = 10240
RPT = N_PAD // NS


def _edge_mlp_body(ea_ref, w1a_ref, b1a_ref, w1b_ref, b1b_ref,
                   hh1_ref, hh2_ref):
    ea = ea_ref[...]
    hh1_ref[...] = jnp.maximum(jnp.dot(ea, w1a_ref[...]) + b1a_ref[...], 0.0)
    hh2_ref[...] = jnp.maximum(jnp.dot(ea, w1b_ref[...]) + b1b_ref[...], 0.0)


def _edge_mlp(ea2p, w1a, b1a, w1b, b1b):
    be = 2048
    grid = (E_PAD // be,)
    return pl.pallas_call(
        _edge_mlp_body,
        grid=grid,
        in_specs=[
            pl.BlockSpec((be, 2), lambda i: (i, 0)),
            pl.BlockSpec((2, HID), lambda i: (0, 0)),
            pl.BlockSpec((1, HID), lambda i: (0, 0)),
            pl.BlockSpec((2, HID), lambda i: (0, 0)),
            pl.BlockSpec((1, HID), lambda i: (0, 0)),
        ],
        out_specs=[
            pl.BlockSpec((be, HID), lambda i: (i, 0)),
            pl.BlockSpec((be, HID), lambda i: (i, 0)),
        ],
        out_shape=[
            jax.ShapeDtypeStruct((E_PAD, HID), jnp.float32),
            jax.ShapeDtypeStruct((E_PAD, HID), jnp.float32),
        ],
    )(ea2p, w1a, b1a, w1b, b1b)


def _node_pre_body(h_ref, a_ref, root_ref, bias_ref, t_ref, r_ref):
    h = h_ref[...]
    t_ref[...] = jnp.dot(h, a_ref[...])
    r_ref[...] = jnp.dot(h, root_ref[...]) + bias_ref[...]


def _node_pre(h, a, root, bias):
    bn = 1000
    d = h.shape[1]
    grid = (N // bn,)
    return pl.pallas_call(
        _node_pre_body,
        grid=grid,
        in_specs=[
            pl.BlockSpec((bn, d), lambda i: (i, 0)),
            pl.BlockSpec((d, 80), lambda i: (0, 0)),
            pl.BlockSpec((d, HID), lambda i: (0, 0)),
            pl.BlockSpec((1, HID), lambda i: (0, 0)),
        ],
        out_specs=[
            pl.BlockSpec((bn, 80), lambda i: (i, 0)),
            pl.BlockSpec((bn, HID), lambda i: (i, 0)),
        ],
        out_shape=[
            jax.ShapeDtypeStruct((N, 80), jnp.float32),
            jax.ShapeDtypeStruct((N, HID), jnp.float32),
        ],
    )(h, a, root, bias)


def _combine_body(a0_ref, a1_ref, r_ref, h_ref):
    h_ref[...] = jnp.maximum(a0_ref[0] + a1_ref[0] + r_ref[...], 0.0)


def _combine(agg, r):
    bn = 1000
    grid = (N // bn,)
    spec = pl.BlockSpec((bn, HID), lambda i: (i, 0))
    return pl.pallas_call(
        _combine_body,
        grid=grid,
        in_specs=[
            pl.BlockSpec((1, bn, HID), lambda i: (0, i, 0)),
            pl.BlockSpec((1, bn, HID), lambda i: (1, i, 0)),
            spec,
        ],
        out_specs=spec,
        out_shape=jax.ShapeDtypeStruct((N, HID), jnp.float32),
    )(agg, agg, r)


_mesh = plsc.VectorSubcoreMesh(core_axis_name="c", subcore_axis_name="s",
                               num_cores=NC, num_subcores=NS)


@functools.partial(
    pl.kernel,
    out_type=jax.ShapeDtypeStruct((NC, N_PAD, HID), jnp.float32),
    mesh=_mesh,
    compiler_params=pltpu.CompilerParams(
        needs_layout_passes=False, use_tc_tiling_on_sc=False),
    scratch_types=[
        pltpu.VMEM((CHUNK,), jnp.int32),
        pltpu.VMEM((CHUNK,), jnp.int32),
        pltpu.VMEM((CHUNK, 80), jnp.float32),
        pltpu.VMEM((CHUNK, HID), jnp.float32),
        pltpu.VMEM((CHUNK, HID), jnp.float32),
        pltpu.VMEM_SHARED((N_PAD, HID), jnp.float32),
        pltpu.SemaphoreType.DMA,
    ],
)
def _edge_pass(t_hbm, hh_hbm, src_hbm, dst_hbm, zero_hbm, out_hbm,
               src_v, dst_v, rows_v, hh_v, msg_v, agg_sh, sem):
    c = lax.axis_index("c")
    s = lax.axis_index("s")
    wid = s * NC + c

    pltpu.sync_copy(zero_hbm.at[pl.ds(s * RPT, RPT)],
                    agg_sh.at[pl.ds(s * RPT, RPT)])
    plsc.subcore_barrier()

    base = wid * EPW

    def chunk(ci, carry):
        off = base + ci * CHUNK
        pltpu.sync_copy(src_hbm.at[pl.ds(off, CHUNK)], src_v)
        pltpu.sync_copy(dst_hbm.at[pl.ds(off, CHUNK)], dst_v)
        pltpu.sync_copy(hh_hbm.at[pl.ds(off, CHUNK)], hh_v)
        pltpu.async_copy(t_hbm.at[src_v], rows_v, sem).wait()
        lanes = lax.iota(jnp.int32, 16)
        for g in range(CHUNK // 16):
            rows = lanes + (g * 16)
            hhg = [plsc.load_gather(hh_v, [rows, jnp.full((16,), h, jnp.int32)])
                   for h in range(HID)]
            for o in range(HID):
                acc = plsc.load_gather(
                    rows_v, [rows, jnp.full((16,), 64 + o, jnp.int32)])
                for h in range(HID):
                    t = plsc.load_gather(
                        rows_v, [rows, jnp.full((16,), h * 8 + o, jnp.int32)])
                    acc = acc + hhg[h] * t
                plsc.store_scatter(msg_v, [rows, jnp.full((16,), o, jnp.int32)],
                                   acc)
        pltpu.sync_copy(msg_v, agg_sh.at[dst_v], add=True)
        return carry

    lax.fori_loop(0, NCHUNK, chunk, 0)
    plsc.subcore_barrier()
    pltpu.sync_copy(agg_sh.at[pl.ds(s * RPT, RPT)],
                    out_hbm.at[c, pl.ds(s * RPT, RPT)])


def _table_weights(w2, b2, in_ch):
    a = w2.reshape(HID, in_ch, HID).transpose(1, 0, 2).reshape(in_ch, 64)
    return jnp.concatenate(
        [a, b2.reshape(in_ch, HID), jnp.zeros((in_ch, 8), jnp.float32)],
        axis=1)


def kernel(x, edge_index, edge_attr, l1_w1, l1_b1, l1_w2, l1_b2, l1_root,
           l1_bias, l2_w1, l2_b1, l2_w2, l2_b2, l2_root, l2_bias):
    ea2 = edge_attr.reshape(E, 2)
    ea2p = jnp.pad(ea2, ((0, E_PAD - E), (0, 0)))
    srcp = jnp.pad(edge_index[0].astype(jnp.int32), (0, E_PAD - E),
                   constant_values=N)
    dstp = jnp.pad(edge_index[1].astype(jnp.int32), (0, E_PAD - E),
                   constant_values=0)

    a1 = _table_weights(l1_w2, l1_b2, IN)
    a2 = _table_weights(l2_w2, l2_b2, HID)
    zero_n8 = jnp.zeros((N_PAD, HID), jnp.float32)

    hh1, hh2 = _edge_mlp(ea2p, l1_w1, l1_b1.reshape(1, HID),
                         l2_w1, l2_b1.reshape(1, HID))

    t1, r1 = _node_pre(x, a1, l1_root, l1_bias.reshape(1, HID))
    agg1 = _edge_pass(jnp.pad(t1, ((0, 8), (0, 0))), hh1, srcp, dstp, zero_n8)
    h1 = _combine(agg1[:, :N], r1)

    t2, r2 = _node_pre(h1, a2, l2_root, l2_bias.reshape(1, HID))
    agg2 = _edge_pass(jnp.pad(t2, ((0, 8), (0, 0))), hh2, srcp, dstp, zero_n8)
    return _combine(agg2[:, :N], r2)


# speedup vs baseline: 2.8349x; 1.3117x over previous
import functools

import jax
import jax.numpy as jnp
from jax import lax
from jax.experimental import pallas as pl
from jax.experimental.pallas import tpu as pltpu
from jax.experimental.pallas import tpu_sc as plsc

N = 10000
E = 160000
IN = 128
HID = 8

NC = 2
NS = 16
NW = NC * NS
CHUNK = 128
E_PAD = 163840
EPW = E_PAD // NW
NCHUNK = EPW // CHUNK
N_PAD = 10240
AGG = N_PAD * HID


def _edge_mlp_body(ea_ref, w1a_ref, b1a_ref, w1b_ref, b1b_ref,
                   hh1_ref, hh2_ref):
    ea = ea_ref[...]
    hh1_ref[...] = jnp.maximum(jnp.dot(ea, w1a_ref[...]) + b1a_ref[...], 0.0)
    hh2_ref[...] = jnp.maximum(jnp.dot(ea, w1b_ref[...]) + b1b_ref[...], 0.0)


def _edge_mlp(ea2p, w1a, b1a, w1b, b1b):
    be = 2048
    grid = (E_PAD // be,)
    return pl.pallas_call(
        _edge_mlp_body,
        grid=grid,
        in_specs=[
            pl.BlockSpec((be, 2), lambda i: (i, 0)),
            pl.BlockSpec((2, HID), lambda i: (0, 0)),
            pl.BlockSpec((1, HID), lambda i: (0, 0)),
            pl.BlockSpec((2, HID), lambda i: (0, 0)),
            pl.BlockSpec((1, HID), lambda i: (0, 0)),
        ],
        out_specs=[
            pl.BlockSpec((be, HID), lambda i: (i, 0)),
            pl.BlockSpec((be, HID), lambda i: (i, 0)),
        ],
        out_shape=[
            jax.ShapeDtypeStruct((E_PAD, HID), jnp.float32),
            jax.ShapeDtypeStruct((E_PAD, HID), jnp.float32),
        ],
    )(ea2p, w1a, b1a, w1b, b1b)


def _node_pre_body(h_ref, a_ref, root_ref, bias_ref, t_ref, r_ref):
    h = h_ref[...]
    t_ref[...] = jnp.dot(h, a_ref[...])
    r_ref[...] = jnp.dot(h, root_ref[...]) + bias_ref[...]


def _node_pre(h, a, root, bias, bn):
    rows = h.shape[0]
    d = h.shape[1]
    grid = (rows // bn,)
    return pl.pallas_call(
        _node_pre_body,
        grid=grid,
        in_specs=[
            pl.BlockSpec((bn, d), lambda i: (i, 0)),
            pl.BlockSpec((d, 80), lambda i: (0, 0)),
            pl.BlockSpec((d, HID), lambda i: (0, 0)),
            pl.BlockSpec((1, HID), lambda i: (0, 0)),
        ],
        out_specs=[
            pl.BlockSpec((bn, 80), lambda i: (i, 0)),
            pl.BlockSpec((bn, HID), lambda i: (i, 0)),
        ],
        out_shape=[
            jax.ShapeDtypeStruct((rows, 80), jnp.float32),
            jax.ShapeDtypeStruct((rows, HID), jnp.float32),
        ],
    )(h, a, root, bias)


def _combine_body(agg_ref, r_ref, h_ref):
    h_ref[...] = jnp.maximum(jnp.sum(agg_ref[...], axis=0) + r_ref[...], 0.0)


def _combine(agg, r_pad):
    bn = 1024
    grid = (N_PAD // bn,)
    return pl.pallas_call(
        _combine_body,
        grid=grid,
        in_specs=[
            pl.BlockSpec((NC, bn, HID), lambda i: (0, i, 0)),
            pl.BlockSpec((bn, HID), lambda i: (i, 0)),
        ],
        out_specs=pl.BlockSpec((bn, HID), lambda i: (i, 0)),
        out_shape=jax.ShapeDtypeStruct((N_PAD, HID), jnp.float32),
    )(agg, r_pad)


_mesh = plsc.VectorSubcoreMesh(core_axis_name="c", subcore_axis_name="s",
                               num_cores=NC, num_subcores=NS)


@functools.partial(
    pl.kernel,
    out_type=jax.ShapeDtypeStruct((NC, N_PAD, HID), jnp.float32),
    mesh=_mesh,
    compiler_params=pltpu.CompilerParams(
        needs_layout_passes=False, use_tc_tiling_on_sc=False),
    scratch_types=[
        pltpu.VMEM((CHUNK,), jnp.int32),
        pltpu.VMEM((CHUNK,), jnp.int32),
        pltpu.VMEM((CHUNK,), jnp.int32),
        pltpu.VMEM((CHUNK,), jnp.int32),
        pltpu.VMEM((CHUNK, HID), jnp.float32),
        pltpu.VMEM((CHUNK, HID), jnp.float32),
        pltpu.VMEM((CHUNK, 80), jnp.float32),
        pltpu.VMEM((CHUNK, 80), jnp.float32),
        pltpu.VMEM((CHUNK, HID), jnp.float32),
        pltpu.VMEM((CHUNK, HID), jnp.float32),
        pltpu.VMEM_SHARED((N_PAD, HID), jnp.float32),
        pltpu.SemaphoreType.DMA,
        pltpu.SemaphoreType.DMA,
        pltpu.SemaphoreType.DMA,
        pltpu.SemaphoreType.DMA,
        pltpu.SemaphoreType.DMA,
        pltpu.SemaphoreType.DMA,
        pltpu.SemaphoreType.DMA,
        pltpu.SemaphoreType.DMA,
    ],
)
def _edge_pass(t_hbm, hh_hbm, src_hbm, dst_hbm, zero_hbm, out_hbm,
               src_v0, src_v1, dst_v0, dst_v1, hh_v0, hh_v1,
               rows_v0, rows_v1, msg_v0, msg_v1, agg_sh,
               sem_i0, sem_i1, sem_d0, sem_d1, sem_g0, sem_g1,
               sem_s0, sem_s1):
    c = lax.axis_index("c")
    s = lax.axis_index("s")
    wid = s * NC + c
    base = wid * EPW
    RPT = N_PAD // NS

    src_v = (src_v0, src_v1)
    dst_v = (dst_v0, dst_v1)
    hh_v = (hh_v0, hh_v1)
    rows_v = (rows_v0, rows_v1)
    msg_v = (msg_v0, msg_v1)
    sem_i = (sem_i0, sem_i1)
    sem_d = (sem_d0, sem_d1)
    sem_g = (sem_g0, sem_g1)
    sem_s = (sem_s0, sem_s1)

    pltpu.sync_copy(zero_hbm.at[pl.ds(s * RPT, RPT)],
                    agg_sh.at[pl.ds(s * RPT, RPT)])
    plsc.subcore_barrier()

    def issue_in(b, ci):
        off = base + ci * CHUNK
        pltpu.async_copy(src_hbm.at[pl.ds(off, CHUNK)], src_v[b], sem_i[b])
        pltpu.async_copy(hh_hbm.at[pl.ds(off, CHUNK)], hh_v[b], sem_i[b])

    def wait_in(b, ci):
        off = base + ci * CHUNK
        pltpu.make_async_copy(src_hbm.at[pl.ds(off, CHUNK)], src_v[b],
                              sem_i[b]).wait()
        pltpu.make_async_copy(hh_hbm.at[pl.ds(off, CHUNK)], hh_v[b],
                              sem_i[b]).wait()

    def issue_dst(b, ci):
        off = base + ci * CHUNK
        pltpu.async_copy(dst_hbm.at[pl.ds(off, CHUNK)], dst_v[b], sem_d[b])

    def wait_dst(b, ci):
        off = base + ci * CHUNK
        pltpu.make_async_copy(dst_hbm.at[pl.ds(off, CHUNK)], dst_v[b],
                              sem_d[b]).wait()

    def issue_gather(b):
        pltpu.async_copy(t_hbm.at[src_v[b]], rows_v[b], sem_g[b])

    def wait_gather(b):
        pltpu.make_async_copy(t_hbm.at[src_v[b]], rows_v[b], sem_g[b]).wait()

    def issue_scatter(b):
        pltpu.async_copy(msg_v[b], agg_sh.at[dst_v[b]], sem_s[b], add=True)

    def wait_scatter(b):
        pltpu.make_async_copy(msg_v[b], agg_sh.at[dst_v[b]], sem_s[b]).wait()

    def compute(b):
        lanes = lax.iota(jnp.int32, 16)
        for g in range(CHUNK // 16):
            rows = lanes + (g * 16)
            hhg = [plsc.load_gather(hh_v[b],
                                    [rows, jnp.full((16,), h, jnp.int32)])
                   for h in range(HID)]
            for o in range(HID):
                acc = plsc.load_gather(
                    rows_v[b], [rows, jnp.full((16,), 64 + o, jnp.int32)])
                for h in range(HID):
                    t = plsc.load_gather(
                        rows_v[b], [rows, jnp.full((16,), h * 8 + o,
                                                   jnp.int32)])
                    acc = acc + hhg[h] * t
                plsc.store_scatter(msg_v[b],
                                   [rows, jnp.full((16,), o, jnp.int32)], acc)

    issue_in(0, 0)
    issue_dst(0, 0)
    wait_in(0, 0)
    issue_gather(0)
    issue_in(1, 1)

    def outer(j, carry):
        for b in range(2):
            ci = j * 2 + b
            nb = 1 - b
            wait_gather(b)

            @pl.when(ci >= 1)
            def _():
                wait_scatter(nb)

            @pl.when(ci + 1 < NCHUNK)
            def _():
                issue_dst(nb, ci + 1)
                wait_in(nb, ci + 1)
                issue_gather(nb)

            compute(b)
            wait_dst(b, ci)
            issue_scatter(b)

            @pl.when(ci + 2 < NCHUNK)
            def _():
                issue_in(b, ci + 2)
        return carry

    lax.fori_loop(0, NCHUNK // 2, outer, 0)
    wait_scatter(1)
    plsc.subcore_barrier()
    pltpu.sync_copy(agg_sh.at[pl.ds(s * RPT, RPT)],
                    out_hbm.at[c, pl.ds(s * RPT, RPT)])


def _table_weights(w2, b2, in_ch):
    a = w2.reshape(HID, in_ch, HID).transpose(1, 0, 2).reshape(in_ch, 64)
    return jnp.concatenate(
        [a, b2.reshape(in_ch, HID), jnp.zeros((in_ch, 8), jnp.float32)],
        axis=1)


def _pad_nodes(t, r):
    pad = N_PAD - N
    return (jnp.pad(t, ((0, pad), (0, 0))), jnp.pad(r, ((0, pad), (0, 0))))


def kernel(x, edge_index, edge_attr, l1_w1, l1_b1, l1_w2, l1_b2, l1_root,
           l1_bias, l2_w1, l2_b1, l2_w2, l2_b2, l2_root, l2_bias):
    ea2 = edge_attr.reshape(E, 2)
    ea2p = jnp.pad(ea2, ((0, E_PAD - E), (0, 0)))
    srcp = jnp.pad(edge_index[0].astype(jnp.int32), (0, E_PAD - E),
                   constant_values=N)
    dstp = jnp.pad(edge_index[1].astype(jnp.int32), (0, E_PAD - E),
                   constant_values=0)

    a1 = _table_weights(l1_w2, l1_b2, IN)
    a2 = _table_weights(l2_w2, l2_b2, HID)
    zero_n8 = jnp.zeros((N_PAD, HID), jnp.float32)

    hh1, hh2 = _edge_mlp(ea2p, l1_w1, l1_b1.reshape(1, HID),
                         l2_w1, l2_b1.reshape(1, HID))

    t1, r1 = _node_pre(x, a1, l1_root, l1_bias.reshape(1, HID), 1000)
    t1p, r1p = _pad_nodes(t1, r1)
    agg1 = _edge_pass(t1p, hh1, srcp, dstp, zero_n8)
    h1 = _combine(agg1, r1p)

    t2, r2 = _node_pre(h1, a2, l2_root, l2_bias.reshape(1, HID), 1024)
    agg2 = _edge_pass(t2, hh2, srcp, dstp, zero_n8)
    h2 = _combine(agg2, r2)
    return h2[:N]


# speedup vs baseline: 3.1915x; 1.1258x over previous
import functools

import jax
import jax.numpy as jnp
from jax import lax
from jax.experimental import pallas as pl
from jax.experimental.pallas import tpu as pltpu
from jax.experimental.pallas import tpu_sc as plsc

N = 10000
E = 160000
IN = 128
HID = 8

NC = 2
NS = 16
NW = NC * NS
CHUNK = 128
E_PAD = 163840
EPW = E_PAD // NW
NCHUNK = EPW // CHUNK
N_PAD = 10240
AGG = N_PAD * HID


def _node_pre_body(h_ref, a_ref, root_ref, bias_ref, t_ref, r_ref):
    h = h_ref[...]
    t_ref[...] = jnp.dot(h, a_ref[...])
    r_ref[...] = jnp.dot(h, root_ref[...]) + bias_ref[...]


def _node_pre(h, a, root, bias, bn):
    rows = h.shape[0]
    d = h.shape[1]
    grid = (rows // bn,)
    return pl.pallas_call(
        _node_pre_body,
        grid=grid,
        in_specs=[
            pl.BlockSpec((bn, d), lambda i: (i, 0)),
            pl.BlockSpec((d, 80), lambda i: (0, 0)),
            pl.BlockSpec((d, HID), lambda i: (0, 0)),
            pl.BlockSpec((1, HID), lambda i: (0, 0)),
        ],
        out_specs=[
            pl.BlockSpec((bn, 80), lambda i: (i, 0)),
            pl.BlockSpec((bn, HID), lambda i: (i, 0)),
        ],
        out_shape=[
            jax.ShapeDtypeStruct((rows, 80), jnp.float32),
            jax.ShapeDtypeStruct((rows, HID), jnp.float32),
        ],
    )(h, a, root, bias)


def _combine_body(agg_ref, r_ref, h_ref):
    h_ref[...] = jnp.maximum(jnp.sum(agg_ref[...], axis=0) + r_ref[...], 0.0)


def _combine(agg, r_pad):
    bn = 1024
    grid = (N_PAD // bn,)
    return pl.pallas_call(
        _combine_body,
        grid=grid,
        in_specs=[
            pl.BlockSpec((NC, bn, HID), lambda i: (0, i, 0)),
            pl.BlockSpec((bn, HID), lambda i: (i, 0)),
        ],
        out_specs=pl.BlockSpec((bn, HID), lambda i: (i, 0)),
        out_shape=jax.ShapeDtypeStruct((N_PAD, HID), jnp.float32),
    )(agg, r_pad)


_mesh = plsc.VectorSubcoreMesh(core_axis_name="c", subcore_axis_name="s",
                               num_cores=NC, num_subcores=NS)


@functools.partial(
    pl.kernel,
    out_type=jax.ShapeDtypeStruct((NC, N_PAD, HID), jnp.float32),
    mesh=_mesh,
    compiler_params=pltpu.CompilerParams(
        needs_layout_passes=False, use_tc_tiling_on_sc=False),
    scratch_types=[
        pltpu.VMEM((CHUNK,), jnp.int32),
        pltpu.VMEM((CHUNK,), jnp.int32),
        pltpu.VMEM((CHUNK,), jnp.int32),
        pltpu.VMEM((CHUNK,), jnp.int32),
        pltpu.VMEM((2 * CHUNK,), jnp.float32),
        pltpu.VMEM((2 * CHUNK,), jnp.float32),
        pltpu.VMEM((24,), jnp.float32),
        pltpu.VMEM((CHUNK, 80), jnp.float32),
        pltpu.VMEM((CHUNK, 80), jnp.float32),
        pltpu.VMEM((CHUNK, HID), jnp.float32),
        pltpu.VMEM((CHUNK, HID), jnp.float32),
        pltpu.VMEM_SHARED((N_PAD, HID), jnp.float32),
        pltpu.SemaphoreType.DMA,
        pltpu.SemaphoreType.DMA,
        pltpu.SemaphoreType.DMA,
        pltpu.SemaphoreType.DMA,
        pltpu.SemaphoreType.DMA,
        pltpu.SemaphoreType.DMA,
        pltpu.SemaphoreType.DMA,
        pltpu.SemaphoreType.DMA,
    ],
)
def _edge_pass(t_hbm, ea_hbm, w_hbm, src_hbm, dst_hbm, zero_hbm, out_hbm,
               src_v0, src_v1, dst_v0, dst_v1, ea_v0, ea_v1, w_v,
               rows_v0, rows_v1, msg_v0, msg_v1, agg_sh,
               sem_i0, sem_i1, sem_d0, sem_d1, sem_g0, sem_g1,
               sem_s0, sem_s1):
    c = lax.axis_index("c")
    s = lax.axis_index("s")
    wid = s * NC + c
    base = wid * EPW
    RPT = N_PAD // NS

    src_v = (src_v0, src_v1)
    dst_v = (dst_v0, dst_v1)
    ea_v = (ea_v0, ea_v1)
    rows_v = (rows_v0, rows_v1)
    msg_v = (msg_v0, msg_v1)
    sem_i = (sem_i0, sem_i1)
    sem_d = (sem_d0, sem_d1)
    sem_g = (sem_g0, sem_g1)
    sem_s = (sem_s0, sem_s1)

    pltpu.sync_copy(zero_hbm.at[pl.ds(s * RPT, RPT)],
                    agg_sh.at[pl.ds(s * RPT, RPT)])
    pltpu.sync_copy(w_hbm, w_v)
    wlo = w_v[pl.ds(0, 16)]
    whi = w_v[pl.ds(8, 16)]
    w1a = [wlo[h] for h in range(HID)]
    w1b = [wlo[HID + h] for h in range(HID)]
    b1 = [whi[HID + h] for h in range(HID)]
    plsc.subcore_barrier()

    def _ea_off(ci):
        off = base + ci * CHUNK
        return jnp.minimum(off, E - CHUNK) * 2

    def issue_in(b, ci):
        off = base + ci * CHUNK
        pltpu.async_copy(src_hbm.at[pl.ds(off, CHUNK)], src_v[b], sem_i[b])
        pltpu.async_copy(ea_hbm.at[pl.ds(_ea_off(ci), 2 * CHUNK)], ea_v[b],
                         sem_i[b])

    def wait_in(b, ci):
        off = base + ci * CHUNK
        pltpu.make_async_copy(src_hbm.at[pl.ds(off, CHUNK)], src_v[b],
                              sem_i[b]).wait()
        pltpu.make_async_copy(ea_hbm.at[pl.ds(_ea_off(ci), 2 * CHUNK)],
                              ea_v[b], sem_i[b]).wait()

    def issue_dst(b, ci):
        off = base + ci * CHUNK
        pltpu.async_copy(dst_hbm.at[pl.ds(off, CHUNK)], dst_v[b], sem_d[b])

    def wait_dst(b, ci):
        off = base + ci * CHUNK
        pltpu.make_async_copy(dst_hbm.at[pl.ds(off, CHUNK)], dst_v[b],
                              sem_d[b]).wait()

    def issue_gather(b):
        pltpu.async_copy(t_hbm.at[src_v[b]], rows_v[b], sem_g[b])

    def wait_gather(b):
        pltpu.make_async_copy(t_hbm.at[src_v[b]], rows_v[b], sem_g[b]).wait()

    def issue_scatter(b):
        pltpu.async_copy(msg_v[b], agg_sh.at[dst_v[b]], sem_s[b], add=True)

    def wait_scatter(b):
        pltpu.make_async_copy(msg_v[b], agg_sh.at[dst_v[b]], sem_s[b]).wait()

    def compute(b):
        lanes = lax.iota(jnp.int32, 16)
        for g in range(CHUNK // 16):
            rows = lanes + (g * 16)
            ea0 = plsc.load_gather(ea_v[b], [rows * 2])
            ea1 = plsc.load_gather(ea_v[b], [rows * 2 + 1])
            hhg = [jnp.maximum(ea0 * w1a[h] + ea1 * w1b[h] + b1[h], 0.0)
                   for h in range(HID)]
            for o in range(HID):
                acc = plsc.load_gather(
                    rows_v[b], [rows, jnp.full((16,), 64 + o, jnp.int32)])
                for h in range(HID):
                    t = plsc.load_gather(
                        rows_v[b], [rows, jnp.full((16,), h * 8 + o,
                                                   jnp.int32)])
                    acc = acc + hhg[h] * t
                plsc.store_scatter(msg_v[b],
                                   [rows, jnp.full((16,), o, jnp.int32)], acc)

    issue_in(0, 0)
    issue_dst(0, 0)
    wait_in(0, 0)
    issue_gather(0)
    issue_in(1, 1)

    def outer(j, carry):
        for b in range(2):
            ci = j * 2 + b
            nb = 1 - b
            wait_gather(b)

            @pl.when(ci >= 1)
            def _():
                wait_scatter(nb)

            @pl.when(ci + 1 < NCHUNK)
            def _():
                issue_dst(nb, ci + 1)
                wait_in(nb, ci + 1)
                issue_gather(nb)

            compute(b)
            wait_dst(b, ci)
            issue_scatter(b)

            @pl.when(ci + 2 < NCHUNK)
            def _():
                issue_in(b, ci + 2)
        return carry

    lax.fori_loop(0, NCHUNK // 2, outer, 0)
    wait_scatter(1)
    plsc.subcore_barrier()
    pltpu.sync_copy(agg_sh.at[pl.ds(s * RPT, RPT)],
                    out_hbm.at[c, pl.ds(s * RPT, RPT)])


def _table_weights(w2, b2, in_ch):
    a = w2.reshape(HID, in_ch, HID).transpose(1, 0, 2).reshape(in_ch, 64)
    return jnp.concatenate(
        [a, b2.reshape(in_ch, HID), jnp.zeros((in_ch, 8), jnp.float32)],
        axis=1)


def _pad_nodes(t, r):
    pad = N_PAD - N
    return (jnp.pad(t, ((0, pad), (0, 0))), jnp.pad(r, ((0, pad), (0, 0))))


def kernel(x, edge_index, edge_attr, l1_w1, l1_b1, l1_w2, l1_b2, l1_root,
           l1_bias, l2_w1, l2_b1, l2_w2, l2_b2, l2_root, l2_bias):
    ea_flat = edge_attr.reshape(2 * E)
    srcp = jnp.pad(edge_index[0].astype(jnp.int32), (0, E_PAD - E),
                   constant_values=N)
    dstp = jnp.pad(edge_index[1].astype(jnp.int32), (0, E_PAD - E),
                   constant_values=0)

    a1 = _table_weights(l1_w2, l1_b2, IN)
    a2 = _table_weights(l2_w2, l2_b2, HID)
    zero_n8 = jnp.zeros((N_PAD, HID), jnp.float32)

    wb1 = jnp.concatenate([l1_w1.reshape(2 * HID), l1_b1])
    wb2 = jnp.concatenate([l2_w1.reshape(2 * HID), l2_b1])

    t1, r1 = _node_pre(x, a1, l1_root, l1_bias.reshape(1, HID), 1000)
    t1p, r1p = _pad_nodes(t1, r1)
    agg1 = _edge_pass(t1p, ea_flat, wb1, srcp, dstp, zero_n8)
    h1 = _combine(agg1, r1p)

    t2, r2 = _node_pre(h1, a2, l2_root, l2_bias.reshape(1, HID), 1024)
    agg2 = _edge_pass(t2, ea_flat, wb2, srcp, dstp, zero_n8)
    h2 = _combine(agg2, r2)
    return h2[:N]


# speedup vs baseline: 3.2975x; 1.0332x over previous
import functools

import jax
import jax.numpy as jnp
from jax import lax
from jax.experimental import pallas as pl
from jax.experimental.pallas import tpu as pltpu
from jax.experimental.pallas import tpu_sc as plsc

N = 10000
E = 160000
IN = 128
HID = 8

NC = 2
NS = 16
NW = NC * NS
CHUNK = 128
E_PAD = 163840
EPW = E_PAD // NW
NCHUNK = EPW // CHUNK
N_PAD = 10240
AGG = N_PAD * HID


def _node_pre_body(h_ref, a_ref, root_ref, bias_ref, t_ref, r_ref):
    h = h_ref[...]
    t_ref[...] = jnp.dot(h, a_ref[...])
    r_ref[...] = jnp.dot(h, root_ref[...]) + bias_ref[...]


def _node_pre(h, a, root, bias, bn):
    rows = h.shape[0]
    d = h.shape[1]
    grid = (rows // bn,)
    return pl.pallas_call(
        _node_pre_body,
        grid=grid,
        in_specs=[
            pl.BlockSpec((bn, d), lambda i: (i, 0)),
            pl.BlockSpec((d, 80), lambda i: (0, 0)),
            pl.BlockSpec((d, HID), lambda i: (0, 0)),
            pl.BlockSpec((1, HID), lambda i: (0, 0)),
        ],
        out_specs=[
            pl.BlockSpec((bn, 80), lambda i: (i, 0)),
            pl.BlockSpec((bn, HID), lambda i: (i, 0)),
        ],
        out_shape=[
            jax.ShapeDtypeStruct((rows, 80), jnp.float32),
            jax.ShapeDtypeStruct((rows, HID), jnp.float32),
        ],
    )(h, a, root, bias)


BE = 2048


def _edge_prep_body(ei_ref, ea_ref, src_ref, dst_ref, ea0_ref, ea1_ref):
    i = pl.program_id(0)
    e0 = i * BE
    gid = e0 + jax.lax.broadcasted_iota(jnp.int32, (BE,), 0)
    valid = gid < E
    src_ref[...] = jnp.where(valid, ei_ref[0, :], N)
    dst_ref[...] = jnp.where(valid, ei_ref[1, :], 0)
    ea = ea_ref[...].reshape(BE, 2)
    ea0_ref[...] = ea[:, 0]
    ea1_ref[...] = ea[:, 1]


def _edge_prep(edge_index, edge_attr):
    grid = (E_PAD // BE,)
    nin = (E - 1) // BE
    return pl.pallas_call(
        _edge_prep_body,
        grid=grid,
        in_specs=[
            pl.BlockSpec((2, BE), lambda i: (0, jnp.minimum(i, nin))),
            pl.BlockSpec((BE, 1, 2), lambda i: (jnp.minimum(i, nin), 0, 0)),
        ],
        out_specs=[
            pl.BlockSpec((BE,), lambda i: (i,)),
            pl.BlockSpec((BE,), lambda i: (i,)),
            pl.BlockSpec((BE,), lambda i: (i,)),
            pl.BlockSpec((BE,), lambda i: (i,)),
        ],
        out_shape=[
            jax.ShapeDtypeStruct((E_PAD,), jnp.int32),
            jax.ShapeDtypeStruct((E_PAD,), jnp.int32),
            jax.ShapeDtypeStruct((E_PAD,), jnp.float32),
            jax.ShapeDtypeStruct((E_PAD,), jnp.float32),
        ],
    )(edge_index, edge_attr)


def _combine_body(agg_ref, r_ref, h_ref):
    h_ref[...] = jnp.maximum(jnp.sum(agg_ref[...], axis=0) + r_ref[...], 0.0)


def _combine(agg, r_pad):
    bn = 1024
    grid = (N_PAD // bn,)
    return pl.pallas_call(
        _combine_body,
        grid=grid,
        in_specs=[
            pl.BlockSpec((NC, bn, HID), lambda i: (0, i, 0)),
            pl.BlockSpec((bn, HID), lambda i: (i, 0)),
        ],
        out_specs=pl.BlockSpec((bn, HID), lambda i: (i, 0)),
        out_shape=jax.ShapeDtypeStruct((N_PAD, HID), jnp.float32),
    )(agg, r_pad)


_mesh = plsc.VectorSubcoreMesh(core_axis_name="c", subcore_axis_name="s",
                               num_cores=NC, num_subcores=NS)


@functools.partial(
    pl.kernel,
    out_type=jax.ShapeDtypeStruct((NC, N_PAD, HID), jnp.float32),
    mesh=_mesh,
    compiler_params=pltpu.CompilerParams(
        needs_layout_passes=False, use_tc_tiling_on_sc=False),
    scratch_types=[
        pltpu.VMEM((CHUNK,), jnp.int32),
        pltpu.VMEM((CHUNK,), jnp.int32),
        pltpu.VMEM((CHUNK,), jnp.int32),
        pltpu.VMEM((CHUNK,), jnp.int32),
        pltpu.VMEM((CHUNK,), jnp.float32),
        pltpu.VMEM((CHUNK,), jnp.float32),
        pltpu.VMEM((CHUNK,), jnp.float32),
        pltpu.VMEM((CHUNK,), jnp.float32),
        pltpu.VMEM((24,), jnp.float32),
        pltpu.VMEM((CHUNK, 80), jnp.float32),
        pltpu.VMEM((CHUNK, 80), jnp.float32),
        pltpu.VMEM((CHUNK, HID), jnp.float32),
        pltpu.VMEM((CHUNK, HID), jnp.float32),
        pltpu.VMEM_SHARED((N_PAD, HID), jnp.float32),
        pltpu.SemaphoreType.DMA,
        pltpu.SemaphoreType.DMA,
        pltpu.SemaphoreType.DMA,
        pltpu.SemaphoreType.DMA,
        pltpu.SemaphoreType.DMA,
        pltpu.SemaphoreType.DMA,
        pltpu.SemaphoreType.DMA,
        pltpu.SemaphoreType.DMA,
    ],
)
def _edge_pass(t_hbm, ea0_hbm, ea1_hbm, w_hbm, src_hbm, dst_hbm, zero_hbm,
               out_hbm,
               src_v0, src_v1, dst_v0, dst_v1, ea0_v0, ea0_v1, ea1_v0,
               ea1_v1, w_v,
               rows_v0, rows_v1, msg_v0, msg_v1, agg_sh,
               sem_i0, sem_i1, sem_d0, sem_d1, sem_g0, sem_g1,
               sem_s0, sem_s1):
    c = lax.axis_index("c")
    s = lax.axis_index("s")
    wid = s * NC + c
    base = wid * EPW
    RPT = N_PAD // NS

    src_v = (src_v0, src_v1)
    dst_v = (dst_v0, dst_v1)
    ea0_v = (ea0_v0, ea0_v1)
    ea1_v = (ea1_v0, ea1_v1)
    rows_v = (rows_v0, rows_v1)
    msg_v = (msg_v0, msg_v1)
    sem_i = (sem_i0, sem_i1)
    sem_d = (sem_d0, sem_d1)
    sem_g = (sem_g0, sem_g1)
    sem_s = (sem_s0, sem_s1)

    pltpu.sync_copy(zero_hbm.at[pl.ds(s * RPT, RPT)],
                    agg_sh.at[pl.ds(s * RPT, RPT)])
    pltpu.sync_copy(w_hbm, w_v)
    wlo = w_v[pl.ds(0, 16)]
    whi = w_v[pl.ds(8, 16)]
    w1a = [wlo[h] for h in range(HID)]
    w1b = [wlo[HID + h] for h in range(HID)]
    b1 = [whi[HID + h] for h in range(HID)]
    plsc.subcore_barrier()

    def issue_in(b, ci):
        off = base + ci * CHUNK
        pltpu.async_copy(src_hbm.at[pl.ds(off, CHUNK)], src_v[b], sem_i[b])
        pltpu.async_copy(ea0_hbm.at[pl.ds(off, CHUNK)], ea0_v[b], sem_i[b])
        pltpu.async_copy(ea1_hbm.at[pl.ds(off, CHUNK)], ea1_v[b], sem_i[b])

    def wait_in(b, ci):
        off = base + ci * CHUNK
        pltpu.make_async_copy(src_hbm.at[pl.ds(off, CHUNK)], src_v[b],
                              sem_i[b]).wait()
        pltpu.make_async_copy(ea0_hbm.at[pl.ds(off, CHUNK)], ea0_v[b],
                              sem_i[b]).wait()
        pltpu.make_async_copy(ea1_hbm.at[pl.ds(off, CHUNK)], ea1_v[b],
                              sem_i[b]).wait()

    def issue_dst(b, ci):
        off = base + ci * CHUNK
        pltpu.async_copy(dst_hbm.at[pl.ds(off, CHUNK)], dst_v[b], sem_d[b])

    def wait_dst(b, ci):
        off = base + ci * CHUNK
        pltpu.make_async_copy(dst_hbm.at[pl.ds(off, CHUNK)], dst_v[b],
                              sem_d[b]).wait()

    def issue_gather(b):
        pltpu.async_copy(t_hbm.at[src_v[b]], rows_v[b], sem_g[b])

    def wait_gather(b):
        pltpu.make_async_copy(t_hbm.at[src_v[b]], rows_v[b], sem_g[b]).wait()

    def issue_scatter(b):
        pltpu.async_copy(msg_v[b], agg_sh.at[dst_v[b]], sem_s[b], add=True)

    def wait_scatter(b):
        pltpu.make_async_copy(msg_v[b], agg_sh.at[dst_v[b]], sem_s[b]).wait()

    def compute(b):
        lanes = lax.iota(jnp.int32, 16)
        for g in range(CHUNK // 16):
            rows = lanes + (g * 16)
            ea0 = plsc.load_gather(ea0_v[b], [rows])
            ea1 = plsc.load_gather(ea1_v[b], [rows])
            hhg = [jnp.maximum(ea0 * w1a[h] + ea1 * w1b[h] + b1[h], 0.0)
                   for h in range(HID)]
            for o in range(HID):
                acc = plsc.load_gather(
                    rows_v[b], [rows, jnp.full((16,), 64 + o, jnp.int32)])
                for h in range(HID):
                    t = plsc.load_gather(
                        rows_v[b], [rows, jnp.full((16,), h * 8 + o,
                                                   jnp.int32)])
                    acc = acc + hhg[h] * t
                plsc.store_scatter(msg_v[b],
                                   [rows, jnp.full((16,), o, jnp.int32)], acc)

    issue_in(0, 0)
    issue_dst(0, 0)
    wait_in(0, 0)
    issue_gather(0)
    issue_in(1, 1)

    def outer(j, carry):
        for b in range(2):
            ci = j * 2 + b
            nb = 1 - b
            wait_gather(b)

            @pl.when(ci >= 1)
            def _():
                wait_scatter(nb)

            @pl.when(ci + 1 < NCHUNK)
            def _():
                issue_dst(nb, ci + 1)
                wait_in(nb, ci + 1)
                issue_gather(nb)

            compute(b)
            wait_dst(b, ci)
            issue_scatter(b)

            @pl.when(ci + 2 < NCHUNK)
            def _():
                issue_in(b, ci + 2)
        return carry

    lax.fori_loop(0, NCHUNK // 2, outer, 0)
    wait_scatter(1)
    plsc.subcore_barrier()
    pltpu.sync_copy(agg_sh.at[pl.ds(s * RPT, RPT)],
                    out_hbm.at[c, pl.ds(s * RPT, RPT)])


def _table_weights(w2, b2, in_ch):
    a = w2.reshape(HID, in_ch, HID).transpose(1, 0, 2).reshape(in_ch, 64)
    return jnp.concatenate(
        [a, b2.reshape(in_ch, HID), jnp.zeros((in_ch, 8), jnp.float32)],
        axis=1)


def _pad_nodes(t, r):
    pad = N_PAD - N
    return (jnp.pad(t, ((0, pad), (0, 0))), jnp.pad(r, ((0, pad), (0, 0))))


def kernel(x, edge_index, edge_attr, l1_w1, l1_b1, l1_w2, l1_b2, l1_root,
           l1_bias, l2_w1, l2_b1, l2_w2, l2_b2, l2_root, l2_bias):
    srcp, dstp, ea0f, ea1f = _edge_prep(edge_index.astype(jnp.int32),
                                        edge_attr)

    a1 = _table_weights(l1_w2, l1_b2, IN)
    a2 = _table_weights(l2_w2, l2_b2, HID)
    zero_n8 = jnp.zeros((N_PAD, HID), jnp.float32)

    wb1 = jnp.concatenate([l1_w1.reshape(2 * HID), l1_b1])
    wb2 = jnp.concatenate([l2_w1.reshape(2 * HID), l2_b1])

    t1, r1 = _node_pre(x, a1, l1_root, l1_bias.reshape(1, HID), 1000)
    t1p, r1p = _pad_nodes(t1, r1)
    agg1 = _edge_pass(t1p, ea0f, ea1f, wb1, srcp, dstp, zero_n8)
    h1 = _combine(agg1, r1p)

    t2, r2 = _node_pre(h1, a2, l2_root, l2_bias.reshape(1, HID), 1024)
    agg2 = _edge_pass(t2, ea0f, ea1f, wb2, srcp, dstp, zero_n8)
    h2 = _combine(agg2, r2)
    return h2[:N]


# speedup vs baseline: 3.6052x; 1.0933x over previous
import functools

import jax
import jax.numpy as jnp
from jax import lax
from jax.experimental import pallas as pl
from jax.experimental.pallas import tpu as pltpu
from jax.experimental.pallas import tpu_sc as plsc

N = 10000
E = 160000
IN = 128
HID = 8

NC = 2
NS = 16
NW = NC * NS
CHUNK = 128
E_PAD = 163840
EPW = E_PAD // NW
NCHUNK = EPW // CHUNK
N_PAD = 10240
AGG = N_PAD * HID


def _node_pre_body(h_ref, a_ref, root_ref, bias_ref, t_ref, r_ref):
    h = h_ref[...]
    t_ref[...] = jnp.dot(h, a_ref[...])
    r_ref[...] = jnp.dot(h, root_ref[...]) + bias_ref[...]


def _node_pre(h, a, root, bias, bn):
    rows = h.shape[0]
    d = h.shape[1]
    grid = (rows // bn,)
    return pl.pallas_call(
        _node_pre_body,
        grid=grid,
        in_specs=[
            pl.BlockSpec((bn, d), lambda i: (i, 0)),
            pl.BlockSpec((d, 80), lambda i: (0, 0)),
            pl.BlockSpec((d, HID), lambda i: (0, 0)),
            pl.BlockSpec((1, HID), lambda i: (0, 0)),
        ],
        out_specs=[
            pl.BlockSpec((bn, 80), lambda i: (i, 0)),
            pl.BlockSpec((bn, HID), lambda i: (i, 0)),
        ],
        out_shape=[
            jax.ShapeDtypeStruct((rows, 80), jnp.float32),
            jax.ShapeDtypeStruct((rows, HID), jnp.float32),
        ],
    )(h, a, root, bias)


BE = 2048


def _edge_prep_body(ei_ref, ea_ref, src_ref, dst_ref, ea0_ref, ea1_ref):
    i = pl.program_id(0)
    e0 = i * BE
    gid = e0 + jax.lax.broadcasted_iota(jnp.int32, (BE,), 0)
    valid = gid < E
    src_ref[...] = jnp.where(valid, ei_ref[0, :], N)
    dst_ref[...] = jnp.where(valid, ei_ref[1, :], 0)
    ea = ea_ref[...].reshape(BE, 2)
    ea0_ref[...] = ea[:, 0]
    ea1_ref[...] = ea[:, 1]


def _edge_prep(edge_index, edge_attr):
    grid = (E_PAD // BE,)
    nin = (E - 1) // BE
    return pl.pallas_call(
        _edge_prep_body,
        grid=grid,
        in_specs=[
            pl.BlockSpec((2, BE), lambda i: (0, jnp.minimum(i, nin))),
            pl.BlockSpec((BE, 1, 2), lambda i: (jnp.minimum(i, nin), 0, 0)),
        ],
        out_specs=[
            pl.BlockSpec((BE,), lambda i: (i,)),
            pl.BlockSpec((BE,), lambda i: (i,)),
            pl.BlockSpec((BE,), lambda i: (i,)),
            pl.BlockSpec((BE,), lambda i: (i,)),
        ],
        out_shape=[
            jax.ShapeDtypeStruct((E_PAD,), jnp.int32),
            jax.ShapeDtypeStruct((E_PAD,), jnp.int32),
            jax.ShapeDtypeStruct((E_PAD,), jnp.float32),
            jax.ShapeDtypeStruct((E_PAD,), jnp.float32),
        ],
    )(edge_index, edge_attr)


def _combine_body(agg_ref, r_ref, h_ref):
    h_ref[...] = jnp.maximum(jnp.sum(agg_ref[...], axis=0) + r_ref[...], 0.0)


def _combine(agg, r_pad):
    bn = 1024
    grid = (N_PAD // bn,)
    return pl.pallas_call(
        _combine_body,
        grid=grid,
        in_specs=[
            pl.BlockSpec((NC, bn, HID), lambda i: (0, i, 0)),
            pl.BlockSpec((bn, HID), lambda i: (i, 0)),
        ],
        out_specs=pl.BlockSpec((bn, HID), lambda i: (i, 0)),
        out_shape=jax.ShapeDtypeStruct((N_PAD, HID), jnp.float32),
    )(agg, r_pad)


_mesh = plsc.VectorSubcoreMesh(core_axis_name="c", subcore_axis_name="s",
                               num_cores=NC, num_subcores=NS)


@functools.partial(
    pl.kernel,
    out_type=jax.ShapeDtypeStruct((NC, N_PAD, HID), jnp.float32),
    mesh=_mesh,
    compiler_params=pltpu.CompilerParams(
        needs_layout_passes=False, use_tc_tiling_on_sc=False),
    scratch_types=[
        [pltpu.VMEM((CHUNK,), jnp.int32) for _ in range(4)],
        [pltpu.VMEM((CHUNK,), jnp.int32) for _ in range(4)],
        [pltpu.VMEM((CHUNK,), jnp.float32) for _ in range(4)],
        [pltpu.VMEM((CHUNK,), jnp.float32) for _ in range(4)],
        [pltpu.VMEM((CHUNK, 80), jnp.float32) for _ in range(4)],
        [pltpu.VMEM((CHUNK, HID), jnp.float32) for _ in range(2)],
        [pltpu.VMEM((CHUNK,), jnp.int32) for _ in range(2)],
        pltpu.VMEM((24,), jnp.float32),
        pltpu.VMEM_SHARED((N_PAD, HID), jnp.float32),
        [pltpu.SemaphoreType.DMA for _ in range(4)],
        [pltpu.SemaphoreType.DMA for _ in range(4)],
        [pltpu.SemaphoreType.DMA for _ in range(2)],
    ],
)
def _edge_pass(t_hbm, ea0_hbm, ea1_hbm, w_hbm, src_hbm, dst_hbm, zero_hbm,
               out_hbm, src_v, dst_v, ea0_v, ea1_v, rows_v, msg_v, dsts_v,
               w_v, agg_sh, sem_i, sem_g, sem_s):
    c = lax.axis_index("c")
    s = lax.axis_index("s")
    wid = s * NC + c
    base = wid * EPW
    RPT = N_PAD // NS

    pltpu.sync_copy(zero_hbm.at[pl.ds(s * RPT, RPT)],
                    agg_sh.at[pl.ds(s * RPT, RPT)])
    pltpu.sync_copy(w_hbm, w_v)
    wlo = w_v[pl.ds(0, 16)]
    whi = w_v[pl.ds(8, 16)]
    w1a = [wlo[h] for h in range(HID)]
    w1b = [wlo[HID + h] for h in range(HID)]
    b1 = [whi[HID + h] for h in range(HID)]
    plsc.subcore_barrier()

    def issue_in(r, ci):
        off = base + ci * CHUNK
        pltpu.async_copy(src_hbm.at[pl.ds(off, CHUNK)], src_v[r], sem_i[r])
        pltpu.async_copy(dst_hbm.at[pl.ds(off, CHUNK)], dst_v[r], sem_i[r])
        pltpu.async_copy(ea0_hbm.at[pl.ds(off, CHUNK)], ea0_v[r], sem_i[r])
        pltpu.async_copy(ea1_hbm.at[pl.ds(off, CHUNK)], ea1_v[r], sem_i[r])

    def wait_in(r, ci):
        off = base + ci * CHUNK
        pltpu.make_async_copy(src_hbm.at[pl.ds(off, CHUNK)], src_v[r],
                              sem_i[r]).wait()
        pltpu.make_async_copy(dst_hbm.at[pl.ds(off, CHUNK)], dst_v[r],
                              sem_i[r]).wait()
        pltpu.make_async_copy(ea0_hbm.at[pl.ds(off, CHUNK)], ea0_v[r],
                              sem_i[r]).wait()
        pltpu.make_async_copy(ea1_hbm.at[pl.ds(off, CHUNK)], ea1_v[r],
                              sem_i[r]).wait()

    def issue_gather(r):
        pltpu.async_copy(t_hbm.at[src_v[r]], rows_v[r], sem_g[r])

    def wait_gather(r):
        pltpu.make_async_copy(t_hbm.at[src_v[r]], rows_v[r], sem_g[r]).wait()

    def issue_scatter(b):
        pltpu.async_copy(msg_v[b], agg_sh.at[dsts_v[b]], sem_s[b], add=True)

    def wait_scatter(b):
        pltpu.make_async_copy(msg_v[b], agg_sh.at[dsts_v[b]],
                              sem_s[b]).wait()

    def compute(r, b):
        lanes = lax.iota(jnp.int32, 16)

        def group(g, carry):
            g16 = g * 16
            rows = lanes + g16
            dsts_v[b][pl.ds(g16, 16)] = dst_v[r][pl.ds(g16, 16)]
            ea0 = plsc.load_gather(ea0_v[r], [rows])
            ea1 = plsc.load_gather(ea1_v[r], [rows])
            hhg = [jnp.maximum(ea0 * w1a[h] + ea1 * w1b[h] + b1[h], 0.0)
                   for h in range(HID)]
            for o in range(HID):
                acc = plsc.load_gather(
                    rows_v[r], [rows, jnp.full((16,), 64 + o, jnp.int32)])
                for h in range(HID):
                    t = plsc.load_gather(
                        rows_v[r], [rows, jnp.full((16,), h * 8 + o,
                                                   jnp.int32)])
                    acc = acc + hhg[h] * t
                plsc.store_scatter(msg_v[b],
                                   [rows, jnp.full((16,), o, jnp.int32)], acc)
            return carry

        lax.fori_loop(0, CHUNK // 16, group, 0)

    for r in range(4):
        issue_in(r, r)
    wait_in(0, 0)
    issue_gather(0)
    wait_in(1, 1)
    issue_gather(1)

    def outer(j, carry):
        for r in range(4):
            ci = j * 4 + r
            b = r % 2
            wait_gather(r)

            @pl.when(ci >= 2)
            def _():
                wait_scatter(b)

            @pl.when(ci + 2 < NCHUNK)
            def _():
                wait_in((r + 2) % 4, ci + 2)
                issue_gather((r + 2) % 4)

            compute(r, b)
            issue_scatter(b)

            @pl.when(ci + 4 < NCHUNK)
            def _():
                issue_in(r, ci + 4)
        return carry

    lax.fori_loop(0, NCHUNK // 4, outer, 0)
    wait_scatter(0)
    wait_scatter(1)
    plsc.subcore_barrier()
    pltpu.sync_copy(agg_sh.at[pl.ds(s * RPT, RPT)],
                    out_hbm.at[c, pl.ds(s * RPT, RPT)])


def _table_weights(w2, b2, in_ch):
    a = w2.reshape(HID, in_ch, HID).transpose(1, 0, 2).reshape(in_ch, 64)
    return jnp.concatenate(
        [a, b2.reshape(in_ch, HID), jnp.zeros((in_ch, 8), jnp.float32)],
        axis=1)


def _pad_nodes(t, r):
    pad = N_PAD - N
    return (jnp.pad(t, ((0, pad), (0, 0))), jnp.pad(r, ((0, pad), (0, 0))))


def kernel(x, edge_index, edge_attr, l1_w1, l1_b1, l1_w2, l1_b2, l1_root,
           l1_bias, l2_w1, l2_b1, l2_w2, l2_b2, l2_root, l2_bias):
    srcp, dstp, ea0f, ea1f = _edge_prep(edge_index.astype(jnp.int32),
                                        edge_attr)

    a1 = _table_weights(l1_w2, l1_b2, IN)
    a2 = _table_weights(l2_w2, l2_b2, HID)
    zero_n8 = jnp.zeros((N_PAD, HID), jnp.float32)

    wb1 = jnp.concatenate([l1_w1.reshape(2 * HID), l1_b1])
    wb2 = jnp.concatenate([l2_w1.reshape(2 * HID), l2_b1])

    t1, r1 = _node_pre(x, a1, l1_root, l1_bias.reshape(1, HID), 1000)
    t1p, r1p = _pad_nodes(t1, r1)
    agg1 = _edge_pass(t1p, ea0f, ea1f, wb1, srcp, dstp, zero_n8)
    h1 = _combine(agg1, r1p)

    t2, r2 = _node_pre(h1, a2, l2_root, l2_bias.reshape(1, HID), 1024)
    agg2 = _edge_pass(t2, ea0f, ea1f, wb2, srcp, dstp, zero_n8)
    h2 = _combine(agg2, r2)
    return h2[:N]


# speedup vs baseline: 4.3912x; 1.2180x over previous
import functools

import jax
import jax.numpy as jnp
from jax import lax
from jax.experimental import pallas as pl
from jax.experimental.pallas import tpu as pltpu
from jax.experimental.pallas import tpu_sc as plsc

N = 10000
E = 160000
IN = 128
HID = 8

NC = 2
NS = 16
NW = NC * NS
CHUNK = 128
E_PAD = 163840
EPW = E_PAD // NW
NCHUNK = EPW // CHUNK
N_PAD = 10240
AGG = N_PAD * HID


def _node_pre_body(h_ref, a_ref, root_ref, bias_ref, t_ref, r_ref):
    h = h_ref[...]
    t_ref[...] = jnp.dot(h, a_ref[...])
    r_ref[...] = jnp.dot(h, root_ref[...]) + bias_ref[...]


def _node_pre(h, a, root, bias, bn):
    rows = h.shape[0]
    d = h.shape[1]
    grid = (rows // bn,)
    return pl.pallas_call(
        _node_pre_body,
        grid=grid,
        in_specs=[
            pl.BlockSpec((bn, d), lambda i: (i, 0)),
            pl.BlockSpec((d, 80), lambda i: (0, 0)),
            pl.BlockSpec((d, HID), lambda i: (0, 0)),
            pl.BlockSpec((1, HID), lambda i: (0, 0)),
        ],
        out_specs=[
            pl.BlockSpec((bn, 80), lambda i: (i, 0)),
            pl.BlockSpec((bn, HID), lambda i: (i, 0)),
        ],
        out_shape=[
            jax.ShapeDtypeStruct((rows, 80), jnp.float32),
            jax.ShapeDtypeStruct((rows, HID), jnp.float32),
        ],
    )(h, a, root, bias)


BE = 2048


def _edge_prep_body(ei_ref, src_ref, dst_ref):
    i = pl.program_id(0)
    gid = i * BE + jax.lax.broadcasted_iota(jnp.int32, (BE,), 0)
    valid = gid < E
    src_ref[...] = jnp.where(valid, ei_ref[0, :], N)
    dst_ref[...] = jnp.where(valid, ei_ref[1, :], 0)


def _edge_prep(edge_index):
    grid = (E_PAD // BE,)
    nin = (E - 1) // BE
    return pl.pallas_call(
        _edge_prep_body,
        grid=grid,
        in_specs=[
            pl.BlockSpec((2, BE), lambda i: (0, jnp.minimum(i, nin))),
        ],
        out_specs=[
            pl.BlockSpec((BE,), lambda i: (i,)),
            pl.BlockSpec((BE,), lambda i: (i,)),
        ],
        out_shape=[
            jax.ShapeDtypeStruct((E_PAD,), jnp.int32),
            jax.ShapeDtypeStruct((E_PAD,), jnp.int32),
        ],
    )(edge_index)


def _combine_body(agg_ref, r_ref, h_ref):
    h_ref[...] = jnp.maximum(jnp.sum(agg_ref[...], axis=0) + r_ref[...], 0.0)


def _combine(agg, r_pad):
    bn = 1024
    grid = (N_PAD // bn,)
    return pl.pallas_call(
        _combine_body,
        grid=grid,
        in_specs=[
            pl.BlockSpec((NC, bn, HID), lambda i: (0, i, 0)),
            pl.BlockSpec((bn, HID), lambda i: (i, 0)),
        ],
        out_specs=pl.BlockSpec((bn, HID), lambda i: (i, 0)),
        out_shape=jax.ShapeDtypeStruct((N_PAD, HID), jnp.float32),
    )(agg, r_pad)


_mesh = plsc.VectorSubcoreMesh(core_axis_name="c", subcore_axis_name="s",
                               num_cores=NC, num_subcores=NS)


@functools.partial(
    pl.kernel,
    out_type=jax.ShapeDtypeStruct((NC, N_PAD, HID), jnp.float32),
    mesh=_mesh,
    compiler_params=pltpu.CompilerParams(
        needs_layout_passes=False, use_tc_tiling_on_sc=False),
    scratch_types=[
        [pltpu.VMEM((CHUNK,), jnp.int32) for _ in range(4)],
        [pltpu.VMEM((CHUNK,), jnp.int32) for _ in range(4)],
        [pltpu.VMEM((2 * CHUNK,), jnp.float32) for _ in range(4)],
        [pltpu.VMEM((CHUNK, 80), jnp.float32) for _ in range(4)],
        [pltpu.VMEM((CHUNK, HID), jnp.float32) for _ in range(2)],
        [pltpu.VMEM((CHUNK,), jnp.int32) for _ in range(2)],
        pltpu.VMEM((24,), jnp.float32),
        pltpu.VMEM_SHARED((N_PAD, HID), jnp.float32),
        [pltpu.SemaphoreType.DMA for _ in range(4)],
        [pltpu.SemaphoreType.DMA for _ in range(4)],
        [pltpu.SemaphoreType.DMA for _ in range(2)],
    ],
)
def _edge_pass(t_hbm, ea_hbm, w_hbm, src_hbm, dst_hbm, zero_hbm,
               out_hbm, src_v, dst_v, ea_v, rows_v, msg_v, dsts_v,
               w_v, agg_sh, sem_i, sem_g, sem_s):
    c = lax.axis_index("c")
    s = lax.axis_index("s")
    base = jnp.where(c == 0, s * (48 * CHUNK),
                     16 * (48 * CHUNK) + s * (32 * CHUNK))
    nch = jnp.where(c == 0, 48, 32)
    RPT = N_PAD // NS

    pltpu.sync_copy(zero_hbm.at[pl.ds(s * RPT, RPT)],
                    agg_sh.at[pl.ds(s * RPT, RPT)])
    pltpu.sync_copy(w_hbm, w_v)
    wlo = w_v[pl.ds(0, 16)]
    whi = w_v[pl.ds(8, 16)]
    w1a = [wlo[h] for h in range(HID)]
    w1b = [wlo[HID + h] for h in range(HID)]
    b1 = [whi[HID + h] for h in range(HID)]
    plsc.subcore_barrier()

    def _ea_off(ci):
        off = base + ci * CHUNK
        return jnp.minimum(off, E - CHUNK) * 2

    def issue_in(r, ci):
        off = base + ci * CHUNK
        pltpu.async_copy(src_hbm.at[pl.ds(off, CHUNK)], src_v[r], sem_i[r])
        pltpu.async_copy(dst_hbm.at[pl.ds(off, CHUNK)], dst_v[r], sem_i[r])
        pltpu.async_copy(ea_hbm.at[pl.ds(_ea_off(ci), 2 * CHUNK)], ea_v[r],
                         sem_i[r])

    def wait_in(r, ci):
        off = base + ci * CHUNK
        pltpu.make_async_copy(src_hbm.at[pl.ds(off, CHUNK)], src_v[r],
                              sem_i[r]).wait()
        pltpu.make_async_copy(dst_hbm.at[pl.ds(off, CHUNK)], dst_v[r],
                              sem_i[r]).wait()
        pltpu.make_async_copy(ea_hbm.at[pl.ds(_ea_off(ci), 2 * CHUNK)],
                              ea_v[r], sem_i[r]).wait()

    def issue_gather(r):
        pltpu.async_copy(t_hbm.at[src_v[r]], rows_v[r], sem_g[r])

    def wait_gather(r):
        pltpu.make_async_copy(t_hbm.at[src_v[r]], rows_v[r], sem_g[r]).wait()

    def issue_scatter(b):
        pltpu.async_copy(msg_v[b], agg_sh.at[dsts_v[b]], sem_s[b], add=True)

    def wait_scatter(b):
        pltpu.make_async_copy(msg_v[b], agg_sh.at[dsts_v[b]],
                              sem_s[b]).wait()

    def compute(r, b):
        lanes = lax.iota(jnp.int32, 16)

        def group(g, carry):
            g16 = g * 16
            rows = lanes + g16
            dsts_v[b][pl.ds(g16, 16)] = dst_v[r][pl.ds(g16, 16)]
            ea0 = plsc.load_gather(ea_v[r], [rows * 2])
            ea1 = plsc.load_gather(ea_v[r], [rows * 2 + 1])
            hhg = [jnp.maximum(ea0 * w1a[h] + ea1 * w1b[h] + b1[h], 0.0)
                   for h in range(HID)]
            for o in range(HID):
                acc = plsc.load_gather(
                    rows_v[r], [rows, jnp.full((16,), 64 + o, jnp.int32)])
                for h in range(HID):
                    t = plsc.load_gather(
                        rows_v[r], [rows, jnp.full((16,), h * 8 + o,
                                                   jnp.int32)])
                    acc = acc + hhg[h] * t
                plsc.store_scatter(msg_v[b],
                                   [rows, jnp.full((16,), o, jnp.int32)], acc)
            return carry

        lax.fori_loop(0, CHUNK // 16, group, 0)

    for r in range(4):
        issue_in(r, r)
    wait_in(0, 0)
    issue_gather(0)
    wait_in(1, 1)
    issue_gather(1)

    def outer(j, carry):
        for r in range(4):
            ci = j * 4 + r
            b = r % 2
            wait_gather(r)

            @pl.when(ci >= 2)
            def _():
                wait_scatter(b)

            @pl.when(ci + 2 < nch)
            def _():
                wait_in((r + 2) % 4, ci + 2)
                issue_gather((r + 2) % 4)

            compute(r, b)
            issue_scatter(b)

            @pl.when(ci + 4 < nch)
            def _():
                issue_in(r, ci + 4)
        return carry

    lax.fori_loop(0, nch // 4, outer, 0)
    wait_scatter(0)
    wait_scatter(1)
    plsc.subcore_barrier()
    pltpu.sync_copy(agg_sh.at[pl.ds(s * RPT, RPT)],
                    out_hbm.at[c, pl.ds(s * RPT, RPT)])


def _table_weights(w2, b2, in_ch):
    a = w2.reshape(HID, in_ch, HID).transpose(1, 0, 2).reshape(in_ch, 64)
    return jnp.concatenate(
        [a, b2.reshape(in_ch, HID), jnp.zeros((in_ch, 8), jnp.float32)],
        axis=1)


def _pad_nodes(t, r):
    pad = N_PAD - N
    return (jnp.pad(t, ((0, pad), (0, 0))), jnp.pad(r, ((0, pad), (0, 0))))


def kernel(x, edge_index, edge_attr, l1_w1, l1_b1, l1_w2, l1_b2, l1_root,
           l1_bias, l2_w1, l2_b1, l2_w2, l2_b2, l2_root, l2_bias):
    srcp, dstp = _edge_prep(edge_index.astype(jnp.int32))
    ea_flat = edge_attr.reshape(2 * E)

    a1 = _table_weights(l1_w2, l1_b2, IN)
    a2 = _table_weights(l2_w2, l2_b2, HID)
    zero_n8 = jnp.zeros((N_PAD, HID), jnp.float32)

    wb1 = jnp.concatenate([l1_w1.reshape(2 * HID), l1_b1])
    wb2 = jnp.concatenate([l2_w1.reshape(2 * HID), l2_b1])

    t1, r1 = _node_pre(x, a1, l1_root, l1_bias.reshape(1, HID), 1000)
    t1p, r1p = _pad_nodes(t1, r1)
    agg1 = _edge_pass(t1p, ea_flat, wb1, srcp, dstp, zero_n8)
    h1 = _combine(agg1, r1p)

    t2, r2 = _node_pre(h1, a2, l2_root, l2_bias.reshape(1, HID), 1024)
    agg2 = _edge_pass(t2, ea_flat, wb2, srcp, dstp, zero_n8)
    h2 = _combine(agg2, r2)
    return h2[:N]
